# Initial kernel scaffold; baseline (speedup 1.0000x reference)
import functools

import jax
import jax.numpy as jnp
from jax import lax
from jax.experimental import pallas as pl
from jax.experimental.pallas import tpu as pltpu
from jax.experimental.pallas import tpu_sc as plsc

NC = 2
NS = 16
CH = 80


def _proj_body(num_rels, num_bases, n_nodes,
               x_ref, basis_ref, wc_ref, lw_ref, src_ref, et_ref,
               proj_ref, selfh_ref, gidx_ref):
  x = x_ref[...]
  t = [jnp.dot(x, basis_ref[b], preferred_element_type=jnp.float32,
               precision=lax.Precision.HIGHEST)
       for b in range(num_bases)]
  for r in range(num_rels):
    acc = t[0] * wc_ref[r, 0]
    for b in range(1, num_bases):
      acc = acc + t[b] * wc_ref[r, b]
    proj_ref[r] = acc
  selfh_ref[...] = jnp.dot(x, lw_ref[...], preferred_element_type=jnp.float32,
                           precision=lax.Precision.HIGHEST)
  gidx_ref[...] = et_ref[...] * jnp.int32(n_nodes) + src_ref[...]


def _final_body(p_ref, selfh_ref, bias_ref, gamma_ref, beta_ref, out_ref):
  h = p_ref[0] + p_ref[1] + selfh_ref[...] + bias_ref[...]
  h = jnp.where(h > 0, h, jnp.float32(0.1) * h)
  mean = jnp.mean(h, axis=-1, keepdims=True)
  d = h - mean
  var = jnp.mean(d * d, axis=-1, keepdims=True)
  out_ref[...] = d * lax.rsqrt(var + jnp.float32(1e-5)) * gamma_ref[...] \
      + beta_ref[...]


def _sc_body(n_nodes, n_chunks_per_worker,
             proj_hbm, gidx_hbm, dst_hbm, zeros_hbm, out_hbm,
             gidx_v, dst_v, rows_v, agg_sh):
  cid = lax.axis_index("c")
  sid = lax.axis_index("s")
  wid = cid * NS + sid
  rows_per_tile = n_nodes // NS

  pltpu.sync_copy(zeros_hbm.at[pl.ds(sid * rows_per_tile, rows_per_tile)],
                  agg_sh.at[pl.ds(sid * rows_per_tile, rows_per_tile)])
  pltpu.sync_copy(gidx_hbm.at[pl.ds(wid * n_chunks_per_worker,
                                    n_chunks_per_worker)], gidx_v)
  pltpu.sync_copy(dst_hbm.at[pl.ds(wid * n_chunks_per_worker,
                                   n_chunks_per_worker)], dst_v)
  plsc.subcore_barrier()

  def body(j, carry):
    pltpu.sync_copy(proj_hbm.at[gidx_v.at[j]], rows_v)
    pltpu.sync_copy(rows_v, agg_sh.at[dst_v.at[j]], add=True)
    return carry

  lax.fori_loop(0, n_chunks_per_worker, body, 0)
  plsc.subcore_barrier()

  pltpu.sync_copy(
      agg_sh.at[pl.ds(sid * rows_per_tile, rows_per_tile)],
      out_hbm.at[pl.ds(cid * n_nodes + sid * rows_per_tile, rows_per_tile)])


def kernel(node_feat, edge_index, edge_types, basis, w_comp, loop_weight,
           bias, ln_gamma, ln_beta):
  n, in_feat = node_feat.shape
  num_bases, _, out_feat = basis.shape
  num_rels = w_comp.shape[0]
  e = edge_index.shape[1]

  src = edge_index[0].astype(jnp.int32)
  dst = edge_index[1].astype(jnp.int32)
  et = edge_types.astype(jnp.int32)

  nb = 1000
  n_blocks = n // nb
  ew = e // n_blocks
  src2d = src.reshape(n_blocks, ew // 128, 128)
  et2d = et.reshape(n_blocks, ew // 128, 128)

  proj, selfh, gidx2d = pl.pallas_call(
      functools.partial(_proj_body, num_rels, num_bases, n),
      grid=(n_blocks,),
      in_specs=[
          pl.BlockSpec((nb, in_feat), lambda i: (i, 0)),
          pl.BlockSpec((num_bases, in_feat, out_feat), lambda i: (0, 0, 0)),
          pl.BlockSpec(memory_space=pltpu.SMEM),
          pl.BlockSpec((in_feat, out_feat), lambda i: (0, 0)),
          pl.BlockSpec((1, ew // 128, 128), lambda i: (i, 0, 0)),
          pl.BlockSpec((1, ew // 128, 128), lambda i: (i, 0, 0)),
      ],
      out_specs=[
          pl.BlockSpec((num_rels, nb, out_feat), lambda i: (0, i, 0)),
          pl.BlockSpec((nb, out_feat), lambda i: (i, 0)),
          pl.BlockSpec((1, ew // 128, 128), lambda i: (i, 0, 0)),
      ],
      out_shape=[
          jax.ShapeDtypeStruct((num_rels, n, out_feat), jnp.float32),
          jax.ShapeDtypeStruct((n, out_feat), jnp.float32),
          jax.ShapeDtypeStruct((n_blocks, ew // 128, 128), jnp.int32),
      ],
  )(node_feat, basis, w_comp, loop_weight, src2d, et2d)

  n_workers = NC * NS
  ncw = e // (n_workers * CH)
  assert e == n_workers * ncw * CH and n % NS == 0
  proj_flat = proj.reshape(num_rels * n, out_feat)
  gidx_c = gidx2d.reshape(n_workers * ncw, CH)
  dst_c = dst.reshape(n_workers * ncw, CH)
  zeros = jnp.zeros((n, out_feat), jnp.float32)

  sc_fn = pl.kernel(
      functools.partial(_sc_body, n, ncw),
      out_type=jax.ShapeDtypeStruct((NC * n, out_feat), jnp.float32),
      mesh=plsc.VectorSubcoreMesh(core_axis_name="c", subcore_axis_name="s"),
      scratch_types=[
          pltpu.VMEM((ncw, CH), jnp.int32),
          pltpu.VMEM((ncw, CH), jnp.int32),
          pltpu.VMEM((CH, out_feat), jnp.float32),
          pltpu.VMEM_SHARED((n, out_feat), jnp.float32),
      ],
  )
  partials = sc_fn(proj_flat, gidx_c, dst_c, zeros).reshape(NC, n, out_feat)

  out = pl.pallas_call(
      _final_body,
      grid=(n_blocks,),
      in_specs=[
          pl.BlockSpec((NC, nb, out_feat), lambda i: (0, i, 0)),
          pl.BlockSpec((nb, out_feat), lambda i: (i, 0)),
          pl.BlockSpec((1, out_feat), lambda i: (0, 0)),
          pl.BlockSpec((1, out_feat), lambda i: (0, 0)),
          pl.BlockSpec((1, out_feat), lambda i: (0, 0)),
      ],
      out_specs=pl.BlockSpec((nb, out_feat), lambda i: (i, 0)),
      out_shape=jax.ShapeDtypeStruct((n, out_feat), jnp.float32),
  )(partials, selfh, bias.reshape(1, out_feat),
    ln_gamma.reshape(1, out_feat), ln_beta.reshape(1, out_feat))
  return out


# speedup vs baseline: 12.6596x; 12.6596x over previous
import functools

import jax
import jax.numpy as jnp
from jax import lax
from jax.experimental import pallas as pl
from jax.experimental.pallas import tpu as pltpu
from jax.experimental.pallas import tpu_sc as plsc

NC = 2
NS = 16
CH = 128


def _proj_body(num_rels, num_bases, n_nodes,
               x_ref, basis_ref, wc_ref, lw_ref, src_ref, et_ref,
               proj_ref, selfh_ref, gidx_ref):
  x = x_ref[...]
  t = [jnp.dot(x, basis_ref[b], preferred_element_type=jnp.float32,
               precision=lax.Precision.HIGHEST)
       for b in range(num_bases)]
  for r in range(num_rels):
    acc = t[0] * wc_ref[r, 0]
    for b in range(1, num_bases):
      acc = acc + t[b] * wc_ref[r, b]
    proj_ref[r] = acc
  selfh_ref[...] = jnp.dot(x, lw_ref[...], preferred_element_type=jnp.float32,
                           precision=lax.Precision.HIGHEST)
  gidx_ref[...] = et_ref[...] * jnp.int32(n_nodes) + src_ref[...]


def _final_body(p_ref, selfh_ref, bias_ref, gamma_ref, beta_ref, out_ref):
  h = p_ref[0] + p_ref[1] + selfh_ref[...] + bias_ref[...]
  h = jnp.where(h > 0, h, jnp.float32(0.1) * h)
  mean = jnp.mean(h, axis=-1, keepdims=True)
  d = h - mean
  var = jnp.mean(d * d, axis=-1, keepdims=True)
  out_ref[...] = d * lax.rsqrt(var + jnp.float32(1e-5)) * gamma_ref[...] \
      + beta_ref[...]


def _sc_body(n_pad, n_chunks_per_worker,
             proj_hbm, gidx_hbm, dst_hbm, zeros_hbm, out_hbm,
             gidx_v, dst_v, rows_v, agg_sh):
  cid = lax.axis_index("c")
  sid = lax.axis_index("s")
  wid = cid * NS + sid
  rows_per_tile = n_pad // NS

  pltpu.sync_copy(zeros_hbm.at[pl.ds(sid * rows_per_tile, rows_per_tile)],
                  agg_sh.at[pl.ds(sid * rows_per_tile, rows_per_tile)])
  pltpu.sync_copy(gidx_hbm.at[wid], gidx_v)
  pltpu.sync_copy(dst_hbm.at[wid], dst_v)
  plsc.subcore_barrier()

  def body(j, carry):
    pltpu.sync_copy(proj_hbm.at[gidx_v.at[j]], rows_v)
    pltpu.sync_copy(rows_v, agg_sh.at[dst_v.at[j]], add=True)
    return carry

  lax.fori_loop(0, n_chunks_per_worker, body, 0)
  plsc.subcore_barrier()

  pltpu.sync_copy(
      agg_sh.at[pl.ds(sid * rows_per_tile, rows_per_tile)],
      out_hbm.at[pl.ds(cid * n_pad + sid * rows_per_tile, rows_per_tile)])


def kernel(node_feat, edge_index, edge_types, basis, w_comp, loop_weight,
           bias, ln_gamma, ln_beta):
  n, in_feat = node_feat.shape
  num_bases, _, out_feat = basis.shape
  num_rels = w_comp.shape[0]
  e = edge_index.shape[1]

  src = edge_index[0].astype(jnp.int32)
  dst = edge_index[1].astype(jnp.int32)
  et = edge_types.astype(jnp.int32)

  nb = 1000
  n_blocks = n // nb
  ew = e // n_blocks
  src2d = src.reshape(n_blocks, ew // 128, 128)
  et2d = et.reshape(n_blocks, ew // 128, 128)

  proj, selfh, gidx2d = pl.pallas_call(
      functools.partial(_proj_body, num_rels, num_bases, n),
      grid=(n_blocks,),
      in_specs=[
          pl.BlockSpec((nb, in_feat), lambda i: (i, 0)),
          pl.BlockSpec((num_bases, in_feat, out_feat), lambda i: (0, 0, 0)),
          pl.BlockSpec(memory_space=pltpu.SMEM),
          pl.BlockSpec((in_feat, out_feat), lambda i: (0, 0)),
          pl.BlockSpec((1, ew // 128, 128), lambda i: (i, 0, 0)),
          pl.BlockSpec((1, ew // 128, 128), lambda i: (i, 0, 0)),
      ],
      out_specs=[
          pl.BlockSpec((num_rels, nb, out_feat), lambda i: (0, i, 0)),
          pl.BlockSpec((nb, out_feat), lambda i: (i, 0)),
          pl.BlockSpec((1, ew // 128, 128), lambda i: (i, 0, 0)),
      ],
      out_shape=[
          jax.ShapeDtypeStruct((num_rels, n, out_feat), jnp.float32),
          jax.ShapeDtypeStruct((n, out_feat), jnp.float32),
          jax.ShapeDtypeStruct((n_blocks, ew // 128, 128), jnp.int32),
      ],
  )(node_feat, basis, w_comp, loop_weight, src2d, et2d)

  n_workers = NC * NS
  npad = ((n + 8 * NS - 1) // (8 * NS)) * (8 * NS)
  ncw = (e + n_workers * CH - 1) // (n_workers * CH)
  e_pad = n_workers * ncw * CH
  proj_flat = proj.reshape(num_rels * n, out_feat)
  gidx_flat = gidx2d.reshape(e)
  pad = e_pad - e
  gidx_c = jnp.concatenate(
      [gidx_flat, jnp.zeros((pad,), jnp.int32)]).reshape(n_workers, ncw, CH)
  dst_c = jnp.concatenate(
      [dst, jnp.full((pad,), n, jnp.int32)]).reshape(n_workers, ncw, CH)
  zeros = jnp.zeros((npad, out_feat), jnp.float32)

  sc_fn = pl.kernel(
      functools.partial(_sc_body, npad, ncw),
      out_type=jax.ShapeDtypeStruct((NC * npad, out_feat), jnp.float32),
      mesh=plsc.VectorSubcoreMesh(core_axis_name="c", subcore_axis_name="s",
                                  num_cores=NC, num_subcores=NS),
      scratch_types=[
          pltpu.VMEM((ncw, CH), jnp.int32),
          pltpu.VMEM((ncw, CH), jnp.int32),
          pltpu.VMEM((CH, out_feat), jnp.float32),
          pltpu.VMEM_SHARED((npad, out_feat), jnp.float32),
      ],
  )
  partials = sc_fn(proj_flat, gidx_c, dst_c, zeros).reshape(NC, npad, out_feat)

  out = pl.pallas_call(
      _final_body,
      grid=(n_blocks,),
      in_specs=[
          pl.BlockSpec((NC, nb, out_feat), lambda i: (0, i, 0)),
          pl.BlockSpec((nb, out_feat), lambda i: (i, 0)),
          pl.BlockSpec((1, out_feat), lambda i: (0, 0)),
          pl.BlockSpec((1, out_feat), lambda i: (0, 0)),
          pl.BlockSpec((1, out_feat), lambda i: (0, 0)),
      ],
      out_specs=pl.BlockSpec((nb, out_feat), lambda i: (i, 0)),
      out_shape=jax.ShapeDtypeStruct((n, out_feat), jnp.float32),
  )(partials, selfh, bias.reshape(1, out_feat),
    ln_gamma.reshape(1, out_feat), ln_beta.reshape(1, out_feat))
  return out
